# Initial kernel scaffold; baseline (speedup 1.0000x reference)
import functools

import jax
import jax.numpy as jnp
from jax import lax
from jax.experimental import pallas as pl
from jax.experimental.pallas import tpu as pltpu
from jax.experimental.pallas import tpu_sc as plsc

_NC = 2
_NS = 16
_NW = _NC * _NS

_CHUNK = 125


def _agg_body(n_chunks, rpt, u_hbm, src_hbm, dst_hbm, out_hbm,
              src_v, dst_v, buf_a, buf_b, acc, sem_a, sem_b):
  c = lax.axis_index("c")
  s = lax.axis_index("s")
  wid = s * _NC + c

  def _zrow(i, carry):
    for j in range(4):
      buf_a[i, pl.ds(j * 16, 16)] = jnp.zeros((16,), jnp.float32)
    return carry
  lax.fori_loop(0, _CHUNK, _zrow, 0)
  for k in range(rpt // _CHUNK):
    pltpu.sync_copy(buf_a, acc.at[pl.ds(s * rpt + k * _CHUNK, _CHUNK)])

  pltpu.sync_copy(src_hbm.at[wid], src_v)
  pltpu.sync_copy(dst_hbm.at[wid], dst_v)
  plsc.subcore_barrier()

  def _step(j, carry):
    pltpu.async_copy(u_hbm.at[src_v.at[j]], buf_a, sem_a).wait()
    pltpu.sync_copy(buf_a, acc.at[dst_v.at[j]], add=True)
    return carry
  lax.fori_loop(0, n_chunks, _step, 0)

  plsc.subcore_barrier()
  pltpu.sync_copy(acc.at[pl.ds(s * rpt, rpt)], out_hbm.at[c, pl.ds(s * rpt, rpt)])


@functools.lru_cache(maxsize=None)
def _make_agg(n, h, e):
  ept = e // _NW
  n_chunks = ept // _CHUNK
  rpt = n // _NS
  mesh = plsc.VectorSubcoreMesh(core_axis_name="c", subcore_axis_name="s")
  return pl.kernel(
      functools.partial(_agg_body, n_chunks, rpt),
      out_type=jax.ShapeDtypeStruct((_NC, n, h), jnp.float32),
      mesh=mesh,
      scratch_types=[
          pltpu.VMEM((n_chunks, _CHUNK), jnp.int32),
          pltpu.VMEM((n_chunks, _CHUNK), jnp.int32),
          pltpu.VMEM((_CHUNK, h), jnp.float32),
          pltpu.VMEM((_CHUNK, h), jnp.float32),
          pltpu.VMEM_SHARED((n, h), jnp.float32),
          pltpu.SemaphoreType.DMA,
          pltpu.SemaphoreType.DMA,
      ])


def _matmul_body(x_ref, w_ref, o_ref):
  o_ref[...] = jnp.dot(x_ref[...], w_ref[...], preferred_element_type=jnp.float32)


def _mid_body(u_ref, parts_ref, scale_ref, b1_ref, w2_ref, b2_ref, w1n_ref, o_ref):
  t = scale_ref[...] * u_ref[...] + parts_ref[0] + parts_ref[1] + b1_ref[...]
  m = jnp.maximum(t, 0.0)
  hh = jnp.dot(m, w2_ref[...], preferred_element_type=jnp.float32) + b2_ref[...]
  hh = jnp.maximum(hh, 0.0)
  o_ref[...] = jnp.dot(hh, w1n_ref[...], preferred_element_type=jnp.float32)


def _final_body(u_ref, parts_ref, scale_ref, b1_ref, w2_ref, b2_ref, batch_ref,
                fc1w_ref, fc1b_ref, fc2w_ref, fc2b_ref, o_ref):
  n = u_ref.shape[0]
  t = scale_ref[...] * u_ref[...] + parts_ref[0] + parts_ref[1] + b1_ref[...]
  m = jnp.maximum(t, 0.0)
  hh = jnp.dot(m, w2_ref[...], preferred_element_type=jnp.float32) + b2_ref[...]
  hh = jnp.maximum(hh, 0.0)
  rows = lax.broadcasted_iota(jnp.int32, (128, n), 0)
  mask = (rows == batch_ref[...]).astype(jnp.float32)
  g = jnp.dot(mask, hh, preferred_element_type=jnp.float32)
  z = jnp.dot(g, fc1w_ref[...], preferred_element_type=jnp.float32) + fc1b_ref[...]
  z = jnp.maximum(z, 0.0)
  o_ref[...] = jnp.dot(z, fc2w_ref[...], preferred_element_type=jnp.float32) + fc2b_ref[...]


def kernel(x, params, edge_index, batch):
  n, _ = x.shape
  e = edge_index.shape[1]
  h = params["conv1"]["W1"].shape[1]

  src3 = edge_index[0].reshape(_NW, -1, _CHUNK)
  dst3 = edge_index[1].reshape(_NW, -1, _CHUNK)
  agg = _make_agg(n, h, e)

  matmul = pl.pallas_call(
      _matmul_body, out_shape=jax.ShapeDtypeStruct((n, h), jnp.float32))
  u = matmul(x, params["conv1"]["W1"])

  for li in range(1, 5):
    p = params["conv%d" % li]
    parts = agg(u, src3, dst3)
    scale = (1.0 + p["eps"]).reshape(1, 1).astype(jnp.float32)
    mid = pl.pallas_call(
        _mid_body, out_shape=jax.ShapeDtypeStruct((n, h), jnp.float32))
    u = mid(u, parts, scale, p["b1"].reshape(1, h), p["W2"],
            p["b2"].reshape(1, h), params["conv%d" % (li + 1)]["W1"])

  p = params["conv5"]
  parts = agg(u, src3, dst3)
  scale = (1.0 + p["eps"]).reshape(1, 1).astype(jnp.float32)
  fin = pl.pallas_call(
      _final_body, out_shape=jax.ShapeDtypeStruct((128, 1), jnp.float32))
  o = fin(u, parts, scale, p["b1"].reshape(1, h), p["W2"], p["b2"].reshape(1, h),
          batch.reshape(1, n).astype(jnp.int32),
          params["fc1"]["W"], params["fc1"]["b"].reshape(1, -1),
          params["fc2"]["W"], params["fc2"]["b"].reshape(1, -1))
  return o[:100]


# speedup vs baseline: 8.0454x; 8.0454x over previous
import functools

import jax
import jax.numpy as jnp
from jax import lax
from jax.experimental import pallas as pl
from jax.experimental.pallas import tpu as pltpu
from jax.experimental.pallas import tpu_sc as plsc

_NC = 2
_NS = 16
_NW = _NC * _NS

_W = 128
_CHUNK = 125
_ZROWS = 80
_STRIPE = 640


def _agg_body(n, n_chunks, u_hbm, src_hbm, dst_hbm, out_hbm,
              src_v, dst_v, buf_a, buf_b, zbuf, acc, sem_a, sem_b):
  c = lax.axis_index("c")
  s = lax.axis_index("s")
  wid = s * _NC + c
  tail = n - (_NS - 1) * _STRIPE

  def _zrow(i, carry):
    for j in range(_W // 16):
      zbuf[i, pl.ds(j * 16, 16)] = jnp.zeros((16,), jnp.float32)
    return carry
  lax.fori_loop(0, _ZROWS, _zrow, 0)

  @pl.when(s < _NS - 1)
  def _():
    for k in range(_STRIPE // _ZROWS):
      pltpu.sync_copy(zbuf, acc.at[pl.ds(s * _STRIPE + k * _ZROWS, _ZROWS)])

  @pl.when(s == _NS - 1)
  def _():
    for k in range(tail // _ZROWS):
      pltpu.sync_copy(zbuf, acc.at[pl.ds(s * _STRIPE + k * _ZROWS, _ZROWS)])

  pltpu.sync_copy(src_hbm.at[wid], src_v)
  pltpu.sync_copy(dst_hbm.at[wid], dst_v)
  plsc.subcore_barrier()

  def _step(j, carry):
    pltpu.async_copy(u_hbm.at[src_v.at[j]], buf_a, sem_a).wait()
    pltpu.sync_copy(buf_a, acc.at[dst_v.at[j]], add=True)
    return carry
  lax.fori_loop(0, n_chunks, _step, 0)

  plsc.subcore_barrier()

  @pl.when(s < _NS - 1)
  def _():
    pltpu.sync_copy(acc.at[pl.ds(s * _STRIPE, _STRIPE)],
                    out_hbm.at[c, pl.ds(s * _STRIPE, _STRIPE)])

  @pl.when(s == _NS - 1)
  def _():
    pltpu.sync_copy(acc.at[pl.ds(s * _STRIPE, tail)],
                    out_hbm.at[c, pl.ds(s * _STRIPE, tail)])


@functools.lru_cache(maxsize=None)
def _make_agg(n, e):
  ept = e // _NW
  n_chunks = ept // _CHUNK
  mesh = plsc.VectorSubcoreMesh(core_axis_name="c", subcore_axis_name="s")
  return pl.kernel(
      functools.partial(_agg_body, n, n_chunks),
      out_type=jax.ShapeDtypeStruct((_NC, n, _W), jnp.float32),
      mesh=mesh,
      scratch_types=[
          pltpu.VMEM((n_chunks, _CHUNK), jnp.int32),
          pltpu.VMEM((n_chunks, _CHUNK), jnp.int32),
          pltpu.VMEM((_CHUNK, _W), jnp.float32),
          pltpu.VMEM((_CHUNK, _W), jnp.float32),
          pltpu.VMEM((_ZROWS, _W), jnp.float32),
          pltpu.VMEM_SHARED((n, _W), jnp.float32),
          pltpu.SemaphoreType.DMA,
          pltpu.SemaphoreType.DMA,
      ])


def _matmul_body(x_ref, w_ref, o_ref):
  o_ref[...] = jnp.dot(x_ref[...], w_ref[...], preferred_element_type=jnp.float32)


def _mid_body(u_ref, parts_ref, scale_ref, b1_ref, w2_ref, b2_ref, w1n_ref, o_ref):
  t = scale_ref[...] * u_ref[...] + parts_ref[0] + parts_ref[1] + b1_ref[...]
  m = jnp.maximum(t, 0.0)
  hh = jnp.dot(m, w2_ref[...], preferred_element_type=jnp.float32) + b2_ref[...]
  hh = jnp.maximum(hh, 0.0)
  o_ref[...] = jnp.dot(hh, w1n_ref[...], preferred_element_type=jnp.float32)


def _final_body(u_ref, parts_ref, scale_ref, b1_ref, w2_ref, b2_ref, batch_ref,
                fc1w_ref, fc1b_ref, fc2w_ref, fc2b_ref, o_ref):
  n = u_ref.shape[0]
  t = scale_ref[...] * u_ref[...] + parts_ref[0] + parts_ref[1] + b1_ref[...]
  m = jnp.maximum(t, 0.0)
  hh = jnp.dot(m, w2_ref[...], preferred_element_type=jnp.float32) + b2_ref[...]
  hh = jnp.maximum(hh, 0.0)
  rows = lax.broadcasted_iota(jnp.int32, (128, n), 0)
  mask = (rows == batch_ref[...]).astype(jnp.float32)
  g = jnp.dot(mask, hh, preferred_element_type=jnp.float32)
  z = jnp.dot(g, fc1w_ref[...], preferred_element_type=jnp.float32) + fc1b_ref[...]
  z = jnp.maximum(z, 0.0)
  o_ref[...] = jnp.dot(z, fc2w_ref[...], preferred_element_type=jnp.float32) + fc2b_ref[...]


def _pad_cols(a, w):
  return jnp.pad(a, ((0, 0), (0, w - a.shape[1])))


def kernel(x, params, edge_index, batch):
  n, _ = x.shape
  e = edge_index.shape[1]
  h = params["conv1"]["W1"].shape[1]

  src3 = edge_index[0].reshape(_NW, -1, _CHUNK)
  dst3 = edge_index[1].reshape(_NW, -1, _CHUNK)
  agg = _make_agg(n, e)

  def w1p(li):
    return _pad_cols(params["conv%d" % li]["W1"], _W)

  def w2p(li):
    p = params["conv%d" % li]
    return jnp.pad(p["W2"], ((0, _W - h), (0, _W - h)))

  matmul = pl.pallas_call(
      _matmul_body, out_shape=jax.ShapeDtypeStruct((n, _W), jnp.float32))
  u = matmul(x, w1p(1))

  for li in range(1, 5):
    p = params["conv%d" % li]
    parts = agg(u, src3, dst3)
    scale = (1.0 + p["eps"]).reshape(1, 1).astype(jnp.float32)
    mid = pl.pallas_call(
        _mid_body, out_shape=jax.ShapeDtypeStruct((n, _W), jnp.float32))
    u = mid(u, parts, scale, _pad_cols(p["b1"].reshape(1, h), _W), w2p(li),
            _pad_cols(p["b2"].reshape(1, h), _W), jnp.pad(w1p(li + 1), ((0, _W - h), (0, 0))))

  p = params["conv5"]
  parts = agg(u, src3, dst3)
  scale = (1.0 + p["eps"]).reshape(1, 1).astype(jnp.float32)
  fin = pl.pallas_call(
      _final_body, out_shape=jax.ShapeDtypeStruct((128, 1), jnp.float32))
  o = fin(u, parts, scale, _pad_cols(p["b1"].reshape(1, h), _W), w2p(5),
          _pad_cols(p["b2"].reshape(1, h), _W),
          batch.reshape(1, n).astype(jnp.int32),
          jnp.pad(params["fc1"]["W"], ((0, _W - h), (0, 0))),
          params["fc1"]["b"].reshape(1, -1),
          params["fc2"]["W"], params["fc2"]["b"].reshape(1, -1))
  return o[:100]


# speedup vs baseline: 8.0525x; 1.0009x over previous
import functools

import jax
import jax.numpy as jnp
from jax import lax
from jax.experimental import pallas as pl
from jax.experimental.pallas import tpu as pltpu
from jax.experimental.pallas import tpu_sc as plsc

_NC = 2
_NS = 16
_NW = _NC * _NS

_W = 128
_H = 64
_CHUNK = 125
_ZROWS = 80
_STRIPE = 640


def _agg_body(n, n_chunks, u_hbm, src_hbm, dst_hbm, out_hbm,
              src_v, dst_v, buf_a, zbuf, acc, sem_a):
  c = lax.axis_index("c")
  s = lax.axis_index("s")
  wid = s * _NC + c
  tail = n - (_NS - 1) * _STRIPE

  def _zrow(i, carry):
    for j in range(_W // 16):
      zbuf[i, pl.ds(j * 16, 16)] = jnp.zeros((16,), jnp.float32)
    return carry
  lax.fori_loop(0, _ZROWS, _zrow, 0)

  @pl.when(s < _NS - 1)
  def _():
    for k in range(_STRIPE // _ZROWS):
      pltpu.sync_copy(zbuf, acc.at[pl.ds(s * _STRIPE + k * _ZROWS, _ZROWS)])

  @pl.when(s == _NS - 1)
  def _():
    for k in range(tail // _ZROWS):
      pltpu.sync_copy(zbuf, acc.at[pl.ds(s * _STRIPE + k * _ZROWS, _ZROWS)])

  pltpu.sync_copy(src_hbm.at[wid], src_v)
  pltpu.sync_copy(dst_hbm.at[wid], dst_v)
  plsc.subcore_barrier()

  def _step(j, carry):
    pltpu.async_copy(u_hbm.at[src_v.at[j]], buf_a, sem_a).wait()
    pltpu.sync_copy(buf_a, acc.at[dst_v.at[j]], add=True)
    return carry
  lax.fori_loop(0, n_chunks, _step, 0)

  plsc.subcore_barrier()

  @pl.when(s < _NS - 1)
  def _():
    pltpu.sync_copy(acc.at[pl.ds(s * _STRIPE, _STRIPE)],
                    out_hbm.at[c, pl.ds(s * _STRIPE, _STRIPE)])

  @pl.when(s == _NS - 1)
  def _():
    pltpu.sync_copy(acc.at[pl.ds(s * _STRIPE, tail)],
                    out_hbm.at[c, pl.ds(s * _STRIPE, tail)])


@functools.lru_cache(maxsize=None)
def _make_agg(n, e):
  ept = e // _NW
  n_chunks = ept // _CHUNK
  mesh = plsc.VectorSubcoreMesh(core_axis_name="c", subcore_axis_name="s")
  return pl.kernel(
      functools.partial(_agg_body, n, n_chunks),
      out_type=jax.ShapeDtypeStruct((_NC, n, _W), jnp.float32),
      mesh=mesh,
      scratch_types=[
          pltpu.VMEM((n_chunks, _CHUNK), jnp.int32),
          pltpu.VMEM((n_chunks, _CHUNK), jnp.int32),
          pltpu.VMEM((_CHUNK, _W), jnp.float32),
          pltpu.VMEM((_ZROWS, _W), jnp.float32),
          pltpu.VMEM_SHARED((n, _W), jnp.float32),
          pltpu.SemaphoreType.DMA,
      ])


def _matmul_body(x_ref, w_ref, o_ref):
  o_ref[...] = jnp.dot(x_ref[...], w_ref[...], preferred_element_type=jnp.float32)


def _mid_body(u_ref, parts_ref, scale_ref, b1_ref, w2_ref, b2_ref, w1n_ref, o_ref):
  t = (scale_ref[...] * u_ref[:, :_H] + parts_ref[0, :, :_H]
       + parts_ref[1, :, :_H] + b1_ref[...])
  m = jnp.maximum(t, 0.0)
  hh = jnp.dot(m, w2_ref[...], preferred_element_type=jnp.float32) + b2_ref[...]
  hh = jnp.maximum(hh, 0.0)
  o_ref[...] = jnp.dot(hh, w1n_ref[...], preferred_element_type=jnp.float32)


def _final_body(u_ref, parts_ref, scale_ref, b1_ref, w2_ref, b2_ref, batch_ref,
                fc1w_ref, fc1b_ref, fc2w_ref, fc2b_ref, o_ref):
  n = u_ref.shape[0]
  t = (scale_ref[...] * u_ref[:, :_H] + parts_ref[0, :, :_H]
       + parts_ref[1, :, :_H] + b1_ref[...])
  m = jnp.maximum(t, 0.0)
  hh = jnp.dot(m, w2_ref[...], preferred_element_type=jnp.float32) + b2_ref[...]
  hh = jnp.maximum(hh, 0.0)
  rows = lax.broadcasted_iota(jnp.int32, (128, n), 0)
  mask = (rows == batch_ref[...]).astype(jnp.float32)
  g = jnp.dot(mask, hh, preferred_element_type=jnp.float32)
  z = jnp.dot(g, fc1w_ref[...], preferred_element_type=jnp.float32) + fc1b_ref[...]
  z = jnp.maximum(z, 0.0)
  o_ref[...] = jnp.dot(z, fc2w_ref[...], preferred_element_type=jnp.float32) + fc2b_ref[...]


def _pad_cols(a, w):
  return jnp.pad(a, ((0, 0), (0, w - a.shape[1])))


def kernel(x, params, edge_index, batch):
  n, _ = x.shape
  e = edge_index.shape[1]
  h = params["conv1"]["W1"].shape[1]

  src3 = edge_index[0].reshape(_NW, -1, _CHUNK)
  dst3 = edge_index[1].reshape(_NW, -1, _CHUNK)
  agg = _make_agg(n, e)

  def w1p(li):
    return _pad_cols(params["conv%d" % li]["W1"], _W)

  matmul = pl.pallas_call(
      _matmul_body, out_shape=jax.ShapeDtypeStruct((n, _W), jnp.float32))
  u = matmul(x, w1p(1))

  for li in range(1, 5):
    p = params["conv%d" % li]
    parts = agg(u, src3, dst3)
    scale = (1.0 + p["eps"]).reshape(1, 1).astype(jnp.float32)
    mid = pl.pallas_call(
        _mid_body, out_shape=jax.ShapeDtypeStruct((n, _W), jnp.float32))
    u = mid(u, parts, scale, p["b1"].reshape(1, h), p["W2"],
            p["b2"].reshape(1, h), w1p(li + 1))

  p = params["conv5"]
  parts = agg(u, src3, dst3)
  scale = (1.0 + p["eps"]).reshape(1, 1).astype(jnp.float32)
  fin = pl.pallas_call(
      _final_body, out_shape=jax.ShapeDtypeStruct((128, 1), jnp.float32))
  o = fin(u, parts, scale, p["b1"].reshape(1, h), p["W2"],
          p["b2"].reshape(1, h),
          batch.reshape(1, n).astype(jnp.int32),
          params["fc1"]["W"], params["fc1"]["b"].reshape(1, -1),
          params["fc2"]["W"], params["fc2"]["b"].reshape(1, -1))
  return o[:100]


# speedup vs baseline: 12.3140x; 1.5292x over previous
import functools

import jax
import jax.numpy as jnp
from jax import lax
from jax.experimental import pallas as pl
from jax.experimental.pallas import tpu as pltpu
from jax.experimental.pallas import tpu_sc as plsc

_NC = 2
_NS = 16
_NW = _NC * _NS

_W = 128
_H = 64
_CHUNK = 125
_ZROWS = 80
_NPHASE = 2
_STRIPE = 640


def _agg_body(n, n_chunks, u_hbm, src_hbm, dst_hbm, out_hbm,
              src_v, dst_v, buf_a, buf_b, acc, sem_a, sem_b):
  c = lax.axis_index("c")
  s = lax.axis_index("s")
  wid = s * _NC + c
  tail = n - (_NS - 1) * _STRIPE

  def _zrow(i, carry):
    for j in range(_W // 16):
      buf_b[i, pl.ds(j * 16, 16)] = jnp.zeros((16,), jnp.float32)
    return carry
  lax.fori_loop(0, _ZROWS, _zrow, 0)

  @pl.when(s < _NS - 1)
  def _():
    for k in range(_STRIPE // _ZROWS):
      pltpu.sync_copy(buf_b.at[pl.ds(0, _ZROWS)],
                      acc.at[pl.ds(s * _STRIPE + k * _ZROWS, _ZROWS)])

  @pl.when(s == _NS - 1)
  def _():
    for k in range(tail // _ZROWS):
      pltpu.sync_copy(buf_b.at[pl.ds(0, _ZROWS)],
                      acc.at[pl.ds(s * _STRIPE + k * _ZROWS, _ZROWS)])

  plsc.subcore_barrier()

  bufs = (buf_a, buf_b)
  sems = (sem_a, sem_b)
  pc = n_chunks // _NPHASE
  for ph in range(_NPHASE):
    pltpu.sync_copy(src_hbm.at[wid, pl.ds(ph * pc, pc)], src_v)
    pltpu.sync_copy(dst_hbm.at[wid, pl.ds(ph * pc, pc)], dst_v)
    pltpu.make_async_copy(u_hbm.at[src_v.at[0]], bufs[0], sems[0]).start()

    def _step(t, carry):
      for b in range(2):
        j = 2 * t + b

        @pl.when(j + 1 < pc)
        def _():
          pltpu.make_async_copy(
              u_hbm.at[src_v.at[j + 1]], bufs[1 - b], sems[1 - b]).start()

        pltpu.make_async_copy(u_hbm.at[src_v.at[j]], bufs[b], sems[b]).wait()
        pltpu.sync_copy(bufs[b], acc.at[dst_v.at[j]], add=True)
      return carry
    lax.fori_loop(0, pc // 2, _step, 0)

  plsc.subcore_barrier()

  @pl.when(s < _NS - 1)
  def _():
    pltpu.sync_copy(acc.at[pl.ds(s * _STRIPE, _STRIPE)],
                    out_hbm.at[c, pl.ds(s * _STRIPE, _STRIPE)])

  @pl.when(s == _NS - 1)
  def _():
    pltpu.sync_copy(acc.at[pl.ds(s * _STRIPE, tail)],
                    out_hbm.at[c, pl.ds(s * _STRIPE, tail)])


@functools.lru_cache(maxsize=None)
def _make_agg(n, e):
  ept = e // _NW
  n_chunks = ept // _CHUNK
  mesh = plsc.VectorSubcoreMesh(core_axis_name="c", subcore_axis_name="s")
  return pl.kernel(
      functools.partial(_agg_body, n, n_chunks),
      out_type=jax.ShapeDtypeStruct((_NC, n, _W), jnp.float32),
      mesh=mesh,
      scratch_types=[
          pltpu.VMEM((n_chunks // _NPHASE, _CHUNK), jnp.int32),
          pltpu.VMEM((n_chunks // _NPHASE, _CHUNK), jnp.int32),
          pltpu.VMEM((_CHUNK, _W), jnp.float32),
          pltpu.VMEM((_CHUNK, _W), jnp.float32),
          pltpu.VMEM_SHARED((n, _W), jnp.float32),
          pltpu.SemaphoreType.DMA,
          pltpu.SemaphoreType.DMA,
      ])


def _matmul_body(x_ref, w_ref, o_ref):
  o_ref[...] = jnp.dot(x_ref[...], w_ref[...], preferred_element_type=jnp.float32)


def _mid_body(u_ref, parts_ref, scale_ref, b1_ref, w2_ref, b2_ref, w1n_ref, o_ref):
  t = (scale_ref[...] * u_ref[:, :_H] + parts_ref[0, :, :_H]
       + parts_ref[1, :, :_H] + b1_ref[...])
  m = jnp.maximum(t, 0.0)
  hh = jnp.dot(m, w2_ref[...], preferred_element_type=jnp.float32) + b2_ref[...]
  hh = jnp.maximum(hh, 0.0)
  o_ref[...] = jnp.dot(hh, w1n_ref[...], preferred_element_type=jnp.float32)


def _final_body(u_ref, parts_ref, scale_ref, b1_ref, w2_ref, b2_ref, batch_ref,
                fc1w_ref, fc1b_ref, fc2w_ref, fc2b_ref, o_ref):
  n = u_ref.shape[0]
  t = (scale_ref[...] * u_ref[:, :_H] + parts_ref[0, :, :_H]
       + parts_ref[1, :, :_H] + b1_ref[...])
  m = jnp.maximum(t, 0.0)
  hh = jnp.dot(m, w2_ref[...], preferred_element_type=jnp.float32) + b2_ref[...]
  hh = jnp.maximum(hh, 0.0)
  rows = lax.broadcasted_iota(jnp.int32, (128, n), 0)
  mask = (rows == batch_ref[...]).astype(jnp.float32)
  g = jnp.dot(mask, hh, preferred_element_type=jnp.float32)
  z = jnp.dot(g, fc1w_ref[...], preferred_element_type=jnp.float32) + fc1b_ref[...]
  z = jnp.maximum(z, 0.0)
  o_ref[...] = jnp.dot(z, fc2w_ref[...], preferred_element_type=jnp.float32) + fc2b_ref[...]


def _pad_cols(a, w):
  return jnp.pad(a, ((0, 0), (0, w - a.shape[1])))


def kernel(x, params, edge_index, batch):
  n, _ = x.shape
  e = edge_index.shape[1]
  h = params["conv1"]["W1"].shape[1]

  src3 = edge_index[0].reshape(_NW, -1, _CHUNK)
  dst3 = edge_index[1].reshape(_NW, -1, _CHUNK)
  agg = _make_agg(n, e)

  def w1p(li):
    return _pad_cols(params["conv%d" % li]["W1"], _W)

  matmul = pl.pallas_call(
      _matmul_body, out_shape=jax.ShapeDtypeStruct((n, _W), jnp.float32))
  u = matmul(x, w1p(1))

  for li in range(1, 5):
    p = params["conv%d" % li]
    parts = agg(u, src3, dst3)
    scale = (1.0 + p["eps"]).reshape(1, 1).astype(jnp.float32)
    mid = pl.pallas_call(
        _mid_body, out_shape=jax.ShapeDtypeStruct((n, _W), jnp.float32))
    u = mid(u, parts, scale, p["b1"].reshape(1, h), p["W2"],
            p["b2"].reshape(1, h), w1p(li + 1))

  p = params["conv5"]
  parts = agg(u, src3, dst3)
  scale = (1.0 + p["eps"]).reshape(1, 1).astype(jnp.float32)
  fin = pl.pallas_call(
      _final_body, out_shape=jax.ShapeDtypeStruct((128, 1), jnp.float32))
  o = fin(u, parts, scale, p["b1"].reshape(1, h), p["W2"],
          p["b2"].reshape(1, h),
          batch.reshape(1, n).astype(jnp.int32),
          params["fc1"]["W"], params["fc1"]["b"].reshape(1, -1),
          params["fc2"]["W"], params["fc2"]["b"].reshape(1, -1))
  return o[:100]
